# baseline (device time: 16833 ns/iter reference)
import jax
import jax.numpy as jnp
from jax import lax
from jax.experimental import pallas as pl
from jax.experimental.pallas import tpu as pltpu

N_DEV = 4
B = 2
SQ = 256
SKV_LOC = 256
HQ = 4
DH = 64
D_MODEL = 512
BLK = 64
SCALE = 0.125
HALVES = 2


def kernel(x, Wq, K_ext, V_ext, Wo):
    def body(x_ref, wq_ref, k_ref, v_ref, wo_ref, out_ref,
             ctx_buf, stat_buf, ctx_sems, st_sems):
        my = lax.axis_index("i")
        left = lax.rem(my + N_DEV - 1, N_DEV)
        right = lax.rem(my + 1, N_DEV)
        opp = lax.rem(my + 2, N_DEV)

        barrier = pltpu.get_barrier_semaphore()
        for nbr in (left, right, opp):
            pl.semaphore_signal(barrier, inc=1, device_id=(nbr,),
                                device_id_type=pl.DeviceIdType.MESH)

        wq_bf = (wq_ref[...] * SCALE).astype(jnp.bfloat16)
        q_both = jnp.dot(x_ref[...].reshape(B * SQ, D_MODEL).astype(jnp.bfloat16),
                         wq_bf, preferred_element_type=jnp.float32)

        targets = ((1, right), (2, left), (3, opp))
        ctx_descs = []
        st_descs = []
        first_send_done = False
        for b in range(B):
            q_all = q_both[b * SQ:(b + 1) * SQ]
            k_all = k_ref[b].reshape(SKV_LOC, HQ * DH).astype(jnp.bfloat16)
            v_all = v_ref[b].reshape(SKV_LOC, HQ * DH).astype(jnp.bfloat16)
            for half in range(HALVES):
                for hh in range(2):
                    h = 2 * half + hh
                    cols = slice(h * DH, (h + 1) * DH)
                    scores = lax.dot_general(
                        q_all[:, cols].astype(jnp.bfloat16), k_all[:, cols],
                        (((1,), (1,)), ((), ())),
                        preferred_element_type=jnp.float32)
                    kept = jnp.concatenate(
                        [scores[j * BLK:(j + 1) * BLK, j * BLK:(j + 1) * BLK]
                         for j in range(SQ // BLK)], axis=0)
                    w = jnp.exp(kept)
                    s = jnp.sum(w, axis=-1, keepdims=True)
                    w_bf = w.astype(jnp.bfloat16)
                    ctx = jnp.concatenate(
                        [jnp.dot(w_bf[j * BLK:(j + 1) * BLK],
                                 v_all[j * BLK:(j + 1) * BLK, cols],
                                 preferred_element_type=jnp.float32)
                         for j in range(SQ // BLK)], axis=0)
                    ctx_buf[0, b, half, :, hh * DH:(hh + 1) * DH] = (
                        ctx.astype(jnp.bfloat16))
                    stat_buf[0, b, h] = s[:, 0]
                if not first_send_done:
                    pl.semaphore_wait(barrier, 3)
                    first_send_done = True
                for t, (dst_slot, dev) in enumerate(targets):
                    si = 12 * b + 6 * half + 2 * t
                    dc = pltpu.make_async_remote_copy(
                        src_ref=ctx_buf.at[0, b, half],
                        dst_ref=ctx_buf.at[dst_slot, b, half],
                        send_sem=ctx_sems.at[si], recv_sem=ctx_sems.at[si + 1],
                        device_id=(dev,), device_id_type=pl.DeviceIdType.MESH)
                    dc.start()
                    ctx_descs.append(dc)
            batch_st = []
            for t, (dst_slot, dev) in enumerate(targets):
                si = 6 * b + 2 * t
                ds = pltpu.make_async_remote_copy(
                    src_ref=stat_buf.at[0, b], dst_ref=stat_buf.at[dst_slot, b],
                    send_sem=st_sems.at[si], recv_sem=st_sems.at[si + 1],
                    device_id=(dev,), device_id_type=pl.DeviceIdType.MESH)
                ds.start()
                batch_st.append(ds)
            st_descs.append(batch_st)

        wo_bf = wo_ref[...].astype(jnp.bfloat16)
        for b in range(B):
            for ds in st_descs[b]:
                ds.wait_recv()
            ssum = (stat_buf[0, b] + stat_buf[1, b]
                    + stat_buf[2, b] + stat_buf[3, b])
            inv_t = jnp.transpose(1.0 / ssum)

            for dc in ctx_descs[6 * b:6 * (b + 1)]:
                dc.wait_recv()
            heads = []
            for h in range(HQ):
                half, off = h // 2, (h % 2) * DH
                acc = (ctx_buf[0, b, half, :, off:off + DH].astype(jnp.float32)
                       + ctx_buf[1, b, half, :, off:off + DH].astype(jnp.float32)
                       + ctx_buf[2, b, half, :, off:off + DH].astype(jnp.float32)
                       + ctx_buf[3, b, half, :, off:off + DH].astype(jnp.float32))
                heads.append(acc * inv_t[:, h:h + 1])
            ctx_full = jnp.concatenate(heads, axis=1)
            out_ref[b] = jnp.dot(ctx_full.astype(jnp.bfloat16), wo_bf,
                                 preferred_element_type=jnp.float32)

        for dc in ctx_descs:
            dc.wait_send()
        for batch_st in st_descs:
            for ds in batch_st:
                ds.wait_send()

    return pl.pallas_call(
        body,
        out_shape=jax.ShapeDtypeStruct((B, SQ, D_MODEL), jnp.float32),
        in_specs=[pl.BlockSpec(memory_space=pltpu.VMEM)] * 5,
        out_specs=pl.BlockSpec(memory_space=pltpu.VMEM),
        scratch_shapes=[
            pltpu.VMEM((N_DEV, B, HALVES, SQ, 2 * DH), jnp.bfloat16),
            pltpu.VMEM((N_DEV, B, HQ, SQ), jnp.float32),
            pltpu.SemaphoreType.DMA((12 * B,)),
            pltpu.SemaphoreType.DMA((6 * B,)),
        ],
        compiler_params=pltpu.CompilerParams(collective_id=0),
    )(x, Wq, K_ext, V_ext, Wo)


# device time: 16384 ns/iter; 1.0274x vs baseline; 1.0274x over previous
import jax
import jax.numpy as jnp
from jax import lax
from jax.experimental import pallas as pl
from jax.experimental.pallas import tpu as pltpu

N_DEV = 4
B = 2
SQ = 256
SKV_LOC = 256
HQ = 4
DH = 64
D_MODEL = 512
BLK = 64
SCALE = 0.125
HALVES = 2


def kernel(x, Wq, K_ext, V_ext, Wo):
    def body(x_ref, wq_ref, k_ref, v_ref, wo_ref, out_ref,
             ctx_buf, stat_buf, ctx_sems, st_sems):
        my = lax.axis_index("i")
        left = lax.rem(my + N_DEV - 1, N_DEV)
        right = lax.rem(my + 1, N_DEV)
        opp = lax.rem(my + 2, N_DEV)

        barrier = pltpu.get_barrier_semaphore()
        for nbr in (left, right, opp):
            pl.semaphore_signal(barrier, inc=1, device_id=(nbr,),
                                device_id_type=pl.DeviceIdType.MESH)

        r = lax.broadcasted_iota(jnp.int32, (SQ, SKV_LOC), 0) // BLK
        c = lax.broadcasted_iota(jnp.int32, (SQ, SKV_LOC), 1) // BLK
        bias = jnp.where(r == c, 0.0, -1e9).astype(jnp.float32)

        wq_bf = (wq_ref[...] * SCALE).astype(jnp.bfloat16)
        q_both = jnp.dot(x_ref[...].reshape(B * SQ, D_MODEL).astype(jnp.bfloat16),
                         wq_bf, preferred_element_type=jnp.float32)

        targets = ((3, opp), (1, right), (2, left))
        ctx_descs = []
        st_descs = []
        first_send_done = False
        for b in range(B):
            q_all = q_both[b * SQ:(b + 1) * SQ]
            k_all = k_ref[b].reshape(SKV_LOC, HQ * DH).astype(jnp.bfloat16)
            v_all = v_ref[b].reshape(SKV_LOC, HQ * DH).astype(jnp.bfloat16)
            for half in range(HALVES):
                for hh in range(2):
                    h = 2 * half + hh
                    cols = slice(h * DH, (h + 1) * DH)
                    scores = lax.dot_general(
                        q_all[:, cols].astype(jnp.bfloat16), k_all[:, cols],
                        (((1,), (1,)), ((), ())),
                        preferred_element_type=jnp.float32) + bias
                    w = jnp.exp(scores)
                    s = jnp.sum(w, axis=-1, keepdims=True)
                    ctx = jnp.dot(w.astype(jnp.bfloat16), v_all[:, cols],
                                  preferred_element_type=jnp.float32)
                    ctx_buf[0, b, half, :, hh * DH:(hh + 1) * DH] = (
                        ctx.astype(jnp.bfloat16))
                    stat_buf[0, b, h] = s[:, 0]
                if not first_send_done:
                    pl.semaphore_wait(barrier, 3)
                    first_send_done = True
                for t, (dst_slot, dev) in enumerate(targets):
                    si = 12 * b + 6 * half + 2 * t
                    dc = pltpu.make_async_remote_copy(
                        src_ref=ctx_buf.at[0, b, half],
                        dst_ref=ctx_buf.at[dst_slot, b, half],
                        send_sem=ctx_sems.at[si], recv_sem=ctx_sems.at[si + 1],
                        device_id=(dev,), device_id_type=pl.DeviceIdType.MESH)
                    dc.start()
                    ctx_descs.append(dc)
            batch_st = []
            for t, (dst_slot, dev) in enumerate(targets):
                si = 6 * b + 2 * t
                ds = pltpu.make_async_remote_copy(
                    src_ref=stat_buf.at[0, b], dst_ref=stat_buf.at[dst_slot, b],
                    send_sem=st_sems.at[si], recv_sem=st_sems.at[si + 1],
                    device_id=(dev,), device_id_type=pl.DeviceIdType.MESH)
                ds.start()
                batch_st.append(ds)
            st_descs.append(batch_st)

        wo_bf = wo_ref[...].astype(jnp.bfloat16)
        for b in range(B):
            for ds in st_descs[b]:
                ds.wait_recv()
            ssum = (stat_buf[0, b] + stat_buf[1, b]
                    + stat_buf[2, b] + stat_buf[3, b])
            inv_t = jnp.transpose(1.0 / ssum)

            for dc in ctx_descs[6 * b:6 * (b + 1)]:
                dc.wait_recv()
            heads = []
            for h in range(HQ):
                half, off = h // 2, (h % 2) * DH
                acc = (ctx_buf[0, b, half, :, off:off + DH].astype(jnp.float32)
                       + ctx_buf[1, b, half, :, off:off + DH].astype(jnp.float32)
                       + ctx_buf[2, b, half, :, off:off + DH].astype(jnp.float32)
                       + ctx_buf[3, b, half, :, off:off + DH].astype(jnp.float32))
                heads.append(acc * inv_t[:, h:h + 1])
            ctx_full = jnp.concatenate(heads, axis=1)
            out_ref[b] = jnp.dot(ctx_full.astype(jnp.bfloat16), wo_bf,
                                 preferred_element_type=jnp.float32)

        for dc in ctx_descs:
            dc.wait_send()
        for batch_st in st_descs:
            for ds in batch_st:
                ds.wait_send()

    return pl.pallas_call(
        body,
        out_shape=jax.ShapeDtypeStruct((B, SQ, D_MODEL), jnp.float32),
        in_specs=[pl.BlockSpec(memory_space=pltpu.VMEM)] * 5,
        out_specs=pl.BlockSpec(memory_space=pltpu.VMEM),
        scratch_shapes=[
            pltpu.VMEM((N_DEV, B, HALVES, SQ, 2 * DH), jnp.bfloat16),
            pltpu.VMEM((N_DEV, B, HQ, SQ), jnp.float32),
            pltpu.SemaphoreType.DMA((12 * B,)),
            pltpu.SemaphoreType.DMA((6 * B,)),
        ],
        compiler_params=pltpu.CompilerParams(collective_id=0),
    )(x, Wq, K_ext, V_ext, Wo)
